# baseline (speedup 1.0000x reference)
import functools

import jax
import jax.numpy as jnp
from jax import lax
from jax.experimental import pallas as pl
from jax.experimental.pallas import tpu as pltpu
from jax.experimental.pallas import tpu_sc as plsc

B = 4
L = 4096
D_MODEL = 768
D_PE = D_MODEL // 2
MAX_LEN = 4096

NUM_CORES = 2
NUM_SUBCORES = 16
NUM_WORKERS = NUM_CORES * NUM_SUBCORES

POS = B * L
POS_PER_WORKER = POS // NUM_WORKERS
WORKERS_PER_BATCH = NUM_WORKERS // B
CHUNK = 16
NCHUNK = POS_PER_WORKER // CHUNK
GROWS = 2 * CHUNK
LANE = 16
COLS = D_PE // LANE
NSLOT = 4


def _pe_add_kernel(f_hbm, idx_hbm, table_hbm, out_hbm,
                   idx_v, f_buf0, f_buf1, f_buf2, f_buf3,
                   pe_buf0, pe_buf1, pe_buf2, pe_buf3,
                   isem, gsem0, gsem1, gsem2, gsem3,
                   fsem0, fsem1, fsem2, fsem3,
                   ssem0, ssem1, ssem2, ssem3):
    f_bufs = (f_buf0, f_buf1, f_buf2, f_buf3)
    pe_bufs = (pe_buf0, pe_buf1, pe_buf2, pe_buf3)
    gsems = (gsem0, gsem1, gsem2, gsem3)
    fsems = (fsem0, fsem1, fsem2, fsem3)
    ssems = (ssem0, ssem1, ssem2, ssem3)

    wid = lax.axis_index("s") * NUM_CORES + lax.axis_index("c")
    b = wid // WORKERS_PER_BATCH
    l0 = (wid % WORKERS_PER_BATCH) * POS_PER_WORKER
    ibase = wid * POS_PER_WORKER * 2
    for c in range(NCHUNK):
        pltpu.async_copy(idx_hbm.at[pl.ds(ibase + c * GROWS, GROWS)],
                         idx_v.at[c], isem)
    for c in range(NCHUNK):
        pltpu.make_async_copy(idx_hbm.at[pl.ds(ibase + c * GROWS, GROWS)],
                              idx_v.at[c], isem).wait()

    def idx_slice(c):
        return idx_v.at[c]

    def f_slice(c):
        return f_hbm.at[b, pl.ds(l0 + c * CHUNK, CHUNK), :]

    def out_slice(c):
        return out_hbm.at[b, pl.ds(l0 + c * CHUNK, CHUNK), :]

    def issue_loads(c, s):
        pltpu.async_copy(table_hbm.at[idx_slice(c)], pe_bufs[s], gsems[s])
        pltpu.async_copy(f_slice(c), f_bufs[s], fsems[s])

    def wait_loads(c, s):
        pltpu.make_async_copy(table_hbm.at[idx_slice(c)], pe_bufs[s],
                              gsems[s]).wait()
        pltpu.make_async_copy(f_slice(c), f_bufs[s], fsems[s]).wait()

    def do_add(s):
        @plsc.parallel_loop(0, CHUNK, 1, unroll=2)
        def _add_pos(r):
            for half in range(2):
                for k in range(COLS):
                    plsc.addupdate(
                        f_bufs[s].at[r, pl.ds(half * D_PE + k * LANE, LANE)],
                        pe_bufs[s][2 * r + half, pl.ds(k * LANE, LANE)])

    def issue_store(c, s):
        pltpu.async_copy(f_bufs[s], out_slice(c), ssems[s])

    def wait_store(c, s):
        pltpu.make_async_copy(f_bufs[s], out_slice(c), ssems[s]).wait()

    issue_loads(0, 0)
    issue_loads(1, 1)
    issue_loads(2, 2)
    wait_loads(0, 0)
    do_add(0)
    issue_store(0, 0)
    issue_loads(3, 3)
    wait_loads(1, 1)
    do_add(1)
    issue_store(1, 1)

    @pl.loop(0, NCHUNK - 4, step=NSLOT)
    def _grp(i):
        for s in range(NSLOT):
            t = i + 4 + s
            wait_store(t - 4, s)
            issue_loads(t, s)
            p = (s + 2) % NSLOT
            wait_loads(t - 2, p)
            do_add(p)
            issue_store(t - 2, p)

    wait_loads(NCHUNK - 2, (NCHUNK - 2) % NSLOT)
    do_add((NCHUNK - 2) % NSLOT)
    issue_store(NCHUNK - 2, (NCHUNK - 2) % NSLOT)
    wait_loads(NCHUNK - 1, (NCHUNK - 1) % NSLOT)
    do_add((NCHUNK - 1) % NSLOT)
    issue_store(NCHUNK - 1, (NCHUNK - 1) % NSLOT)
    for s in range(NSLOT):
        wait_store(NCHUNK - 4 + s, (NCHUNK - 4 + s) % NSLOT)


@jax.jit
def _pe_add(f, idx, table):
    mesh = plsc.VectorSubcoreMesh(core_axis_name="c", subcore_axis_name="s")
    return pl.kernel(
        _pe_add_kernel,
        out_type=jax.ShapeDtypeStruct((B, L, D_MODEL), jnp.float32),
        mesh=mesh,
        scratch_types=(
            [pltpu.VMEM((NCHUNK, GROWS), jnp.int32)]
            + [pltpu.VMEM((CHUNK, D_MODEL), jnp.float32)] * NSLOT
            + [pltpu.VMEM((GROWS, D_PE), jnp.float32)] * NSLOT
            + [pltpu.SemaphoreType.DMA] * (1 + 3 * NSLOT)
        ),
    )(f, idx, table)


def kernel(f, rank, pe):
    table = pe.reshape(MAX_LEN, D_PE)
    idx = rank.astype(jnp.int32).reshape(POS * 2)
    return _pe_add(f, idx, table)


# speedup vs baseline: 1.0854x; 1.0854x over previous
import functools

import jax
import jax.numpy as jnp
from jax import lax
from jax.experimental import pallas as pl
from jax.experimental.pallas import tpu as pltpu
from jax.experimental.pallas import tpu_sc as plsc

B = 4
L = 4096
D_MODEL = 768
D_PE = D_MODEL // 2
MAX_LEN = 4096

NUM_CORES = 2
NUM_SUBCORES = 16
NUM_WORKERS = NUM_CORES * NUM_SUBCORES

POS = B * L
POS_PER_WORKER = POS // NUM_WORKERS
WORKERS_PER_BATCH = NUM_WORKERS // B
CHUNK = 16
NCHUNK = POS_PER_WORKER // CHUNK
GROWS = 2 * CHUNK
LANE = 16
COLS = D_PE // LANE
NSLOT = 4


def _pe_add_kernel(f_hbm, idx_hbm, table_hbm, out_hbm,
                   idx_v, f_buf0, f_buf1, f_buf2, f_buf3,
                   pe_buf0, pe_buf1, pe_buf2, pe_buf3,
                   isem, gsem0, gsem1, gsem2, gsem3,
                   fsem0, fsem1, fsem2, fsem3,
                   ssem0, ssem1, ssem2, ssem3):
    f_bufs = (f_buf0, f_buf1, f_buf2, f_buf3)
    pe_bufs = (pe_buf0, pe_buf1, pe_buf2, pe_buf3)
    gsems = (gsem0, gsem1, gsem2, gsem3)
    fsems = (fsem0, fsem1, fsem2, fsem3)
    ssems = (ssem0, ssem1, ssem2, ssem3)

    wid = lax.axis_index("s") * NUM_CORES + lax.axis_index("c")
    b = wid // WORKERS_PER_BATCH
    l0 = (wid % WORKERS_PER_BATCH) * POS_PER_WORKER
    ibase = wid * POS_PER_WORKER * 2
    for c in range(NCHUNK):
        pltpu.async_copy(idx_hbm.at[pl.ds(ibase + c * GROWS, GROWS)],
                         idx_v.at[c], isem)
    for c in range(NCHUNK):
        pltpu.make_async_copy(idx_hbm.at[pl.ds(ibase + c * GROWS, GROWS)],
                              idx_v.at[c], isem).wait()

    def idx_slice(c):
        return idx_v.at[c]

    def f_slice(c):
        return f_hbm.at[b, pl.ds(l0 + c * CHUNK, CHUNK), :]

    def out_slice(c):
        return out_hbm.at[b, pl.ds(l0 + c * CHUNK, CHUNK), :]

    def issue_loads(c, s):
        pltpu.async_copy(table_hbm.at[idx_slice(c)], pe_bufs[s], gsems[s])
        pltpu.async_copy(f_slice(c), f_bufs[s], fsems[s])

    def wait_loads(c, s):
        pltpu.make_async_copy(table_hbm.at[idx_slice(c)], pe_bufs[s],
                              gsems[s]).wait()
        pltpu.make_async_copy(f_slice(c), f_bufs[s], fsems[s]).wait()

    def do_add(s):
        @plsc.parallel_loop(0, CHUNK, 1, unroll=1)
        def _add_pos(r):
            for half in range(2):
                for k in range(COLS):
                    plsc.addupdate(
                        f_bufs[s].at[r, pl.ds(half * D_PE + k * LANE, LANE)],
                        pe_bufs[s][2 * r + half, pl.ds(k * LANE, LANE)])

    def issue_store(c, s):
        pltpu.async_copy(f_bufs[s], out_slice(c), ssems[s])

    def wait_store(c, s):
        pltpu.make_async_copy(f_bufs[s], out_slice(c), ssems[s]).wait()

    issue_loads(0, 0)
    issue_loads(1, 1)
    issue_loads(2, 2)
    wait_loads(0, 0)
    do_add(0)
    issue_store(0, 0)
    issue_loads(3, 3)
    wait_loads(1, 1)
    do_add(1)
    issue_store(1, 1)

    @pl.loop(0, NCHUNK - 4, step=NSLOT)
    def _grp(i):
        for s in range(NSLOT):
            t = i + 4 + s
            wait_store(t - 4, s)
            issue_loads(t, s)
            p = (s + 2) % NSLOT
            wait_loads(t - 2, p)
            do_add(p)
            issue_store(t - 2, p)

    wait_loads(NCHUNK - 2, (NCHUNK - 2) % NSLOT)
    do_add((NCHUNK - 2) % NSLOT)
    issue_store(NCHUNK - 2, (NCHUNK - 2) % NSLOT)
    wait_loads(NCHUNK - 1, (NCHUNK - 1) % NSLOT)
    do_add((NCHUNK - 1) % NSLOT)
    issue_store(NCHUNK - 1, (NCHUNK - 1) % NSLOT)
    for s in range(NSLOT):
        wait_store(NCHUNK - 4 + s, (NCHUNK - 4 + s) % NSLOT)


@jax.jit
def _pe_add(f, idx, table):
    mesh = plsc.VectorSubcoreMesh(core_axis_name="c", subcore_axis_name="s")
    return pl.kernel(
        _pe_add_kernel,
        out_type=jax.ShapeDtypeStruct((B, L, D_MODEL), jnp.float32),
        mesh=mesh,
        scratch_types=(
            [pltpu.VMEM((NCHUNK, GROWS), jnp.int32)]
            + [pltpu.VMEM((CHUNK, D_MODEL), jnp.float32)] * NSLOT
            + [pltpu.VMEM((GROWS, D_PE), jnp.float32)] * NSLOT
            + [pltpu.SemaphoreType.DMA] * (1 + 3 * NSLOT)
        ),
    )(f, idx, table)


def kernel(f, rank, pe):
    table = pe.reshape(MAX_LEN, D_PE)
    idx = rank.astype(jnp.int32).reshape(POS * 2)
    return _pe_add(f, idx, table)


# speedup vs baseline: 1.1629x; 1.0714x over previous
import functools

import jax
import jax.numpy as jnp
from jax import lax
from jax.experimental import pallas as pl
from jax.experimental.pallas import tpu as pltpu
from jax.experimental.pallas import tpu_sc as plsc

B = 4
L = 4096
D_MODEL = 768
D_PE = D_MODEL // 2
MAX_LEN = 4096

NUM_CORES = 2
NUM_SUBCORES = 16
NUM_WORKERS = NUM_CORES * NUM_SUBCORES

POS = B * L
POS_PER_WORKER = POS // NUM_WORKERS
WORKERS_PER_BATCH = NUM_WORKERS // B
CHUNK = 16
NCHUNK = POS_PER_WORKER // CHUNK
GROWS = 2 * CHUNK
LANE = 16
COLS = D_PE // LANE
NSLOT = 4


def _pe_add_kernel(f_hbm, idx_hbm, table_hbm, out_hbm,
                   idx_v, f_buf0, f_buf1, f_buf2, f_buf3,
                   pe_buf0, pe_buf1, pe_buf2, pe_buf3,
                   gsem0, gsem1, gsem2, gsem3,
                   fsem0, fsem1, fsem2, fsem3,
                   ssem0, ssem1, ssem2, ssem3):
    f_bufs = (f_buf0, f_buf1, f_buf2, f_buf3)
    pe_bufs = (pe_buf0, pe_buf1, pe_buf2, pe_buf3)
    gsems = (gsem0, gsem1, gsem2, gsem3)
    fsems = (fsem0, fsem1, fsem2, fsem3)
    ssems = (ssem0, ssem1, ssem2, ssem3)

    wid = lax.axis_index("s") * NUM_CORES + lax.axis_index("c")
    b = wid // WORKERS_PER_BATCH
    l0 = (wid % WORKERS_PER_BATCH) * POS_PER_WORKER
    pltpu.sync_copy(idx_hbm.at[b, pl.ds(l0 * 2, POS_PER_WORKER * 2)], idx_v)

    def idx_slice(c):
        return idx_v.at[pl.ds(c * GROWS, GROWS)]

    def f_slice(c):
        return f_hbm.at[b, pl.ds(l0 + c * CHUNK, CHUNK), :]

    def out_slice(c):
        return out_hbm.at[b, pl.ds(l0 + c * CHUNK, CHUNK), :]

    def issue_loads(c, s):
        pltpu.async_copy(table_hbm.at[idx_slice(c)], pe_bufs[s], gsems[s])
        pltpu.async_copy(f_slice(c), f_bufs[s], fsems[s])

    def wait_loads(c, s):
        pltpu.make_async_copy(table_hbm.at[idx_slice(c)], pe_bufs[s],
                              gsems[s]).wait()
        pltpu.make_async_copy(f_slice(c), f_bufs[s], fsems[s]).wait()

    def do_add(s):
        @plsc.parallel_loop(0, CHUNK, 1, unroll=1)
        def _add_pos(r):
            for half in range(2):
                for k in range(COLS):
                    plsc.addupdate(
                        f_bufs[s].at[r, pl.ds(half * D_PE + k * LANE, LANE)],
                        pe_bufs[s][2 * r + half, pl.ds(k * LANE, LANE)])

    def issue_store(c, s):
        pltpu.async_copy(f_bufs[s], out_slice(c), ssems[s])

    def wait_store(c, s):
        pltpu.make_async_copy(f_bufs[s], out_slice(c), ssems[s]).wait()

    issue_loads(0, 0)
    issue_loads(1, 1)
    issue_loads(2, 2)
    wait_loads(0, 0)
    do_add(0)
    issue_store(0, 0)
    issue_loads(3, 3)
    wait_loads(1, 1)
    do_add(1)
    issue_store(1, 1)

    @pl.loop(0, NCHUNK - 4, step=NSLOT)
    def _grp(i):
        for s in range(NSLOT):
            t = i + 4 + s
            wait_store(t - 4, s)
            issue_loads(t, s)
            p = (s + 2) % NSLOT
            wait_loads(t - 2, p)
            do_add(p)
            issue_store(t - 2, p)

    wait_loads(NCHUNK - 2, (NCHUNK - 2) % NSLOT)
    do_add((NCHUNK - 2) % NSLOT)
    issue_store(NCHUNK - 2, (NCHUNK - 2) % NSLOT)
    wait_loads(NCHUNK - 1, (NCHUNK - 1) % NSLOT)
    do_add((NCHUNK - 1) % NSLOT)
    issue_store(NCHUNK - 1, (NCHUNK - 1) % NSLOT)
    for s in range(NSLOT):
        wait_store(NCHUNK - 4 + s, (NCHUNK - 4 + s) % NSLOT)


@jax.jit
def _pe_add(f, idx, table):
    mesh = plsc.VectorSubcoreMesh(core_axis_name="c", subcore_axis_name="s")
    return pl.kernel(
        _pe_add_kernel,
        out_type=jax.ShapeDtypeStruct((B, L, D_MODEL), jnp.float32),
        mesh=mesh,
        scratch_types=(
            [pltpu.VMEM((POS_PER_WORKER * 2,), jnp.int32)]
            + [pltpu.VMEM((CHUNK, D_MODEL), jnp.float32)] * NSLOT
            + [pltpu.VMEM((GROWS, D_PE), jnp.float32)] * NSLOT
            + [pltpu.SemaphoreType.DMA] * (3 * NSLOT)
        ),
    )(f, idx, table)


def kernel(f, rank, pe):
    idx = rank.astype(jnp.int32).reshape(B, L * 2)
    table = pe.reshape(MAX_LEN, D_PE)
    return _pe_add(f, idx, table)


# speedup vs baseline: 1.1787x; 1.0136x over previous
import functools

import jax
import jax.numpy as jnp
from jax import lax
from jax.experimental import pallas as pl
from jax.experimental.pallas import tpu as pltpu
from jax.experimental.pallas import tpu_sc as plsc

B = 4
L = 4096
D_MODEL = 768
D_PE = D_MODEL // 2
MAX_LEN = 4096

NUM_CORES = 2
NUM_SUBCORES = 16
NUM_WORKERS = NUM_CORES * NUM_SUBCORES

POS = B * L
POS_PER_WORKER = POS // NUM_WORKERS
WORKERS_PER_BATCH = NUM_WORKERS // B
CHUNK = 32
NCHUNK = POS_PER_WORKER // CHUNK
GROWS = 2 * CHUNK
LANE = 16
COLS = D_PE // LANE
NSLOT = 2
LOOKAHEAD = NSLOT // 2


def _pe_add_kernel(f_hbm, idx_hbm, table_hbm, out_hbm, idx_v, *rest):
    f_bufs = rest[:NSLOT]
    pe_bufs = rest[NSLOT:2 * NSLOT]
    gsems = rest[2 * NSLOT:3 * NSLOT]
    fsems = rest[3 * NSLOT:4 * NSLOT]
    ssems = rest[4 * NSLOT:5 * NSLOT]

    wid = lax.axis_index("s") * NUM_CORES + lax.axis_index("c")
    b = wid // WORKERS_PER_BATCH
    l0 = (wid % WORKERS_PER_BATCH) * POS_PER_WORKER
    pltpu.sync_copy(idx_hbm.at[b, pl.ds(l0 * 2, POS_PER_WORKER * 2)], idx_v)

    def idx_slice(c):
        return idx_v.at[pl.ds(c * GROWS, GROWS)]

    def f_slice(c):
        return f_hbm.at[b, pl.ds(l0 + c * CHUNK, CHUNK), :]

    def out_slice(c):
        return out_hbm.at[b, pl.ds(l0 + c * CHUNK, CHUNK), :]

    def issue_loads(c, s):
        pltpu.async_copy(table_hbm.at[idx_slice(c)], pe_bufs[s], gsems[s])
        pltpu.async_copy(f_slice(c), f_bufs[s], fsems[s])

    def wait_loads(c, s):
        pltpu.make_async_copy(table_hbm.at[idx_slice(c)], pe_bufs[s],
                              gsems[s]).wait()
        pltpu.make_async_copy(f_slice(c), f_bufs[s], fsems[s]).wait()

    def do_add(s):
        @plsc.parallel_loop(0, CHUNK, 1, unroll=1)
        def _add_pos(r):
            for half in range(2):
                for k in range(COLS):
                    plsc.addupdate(
                        f_bufs[s].at[r, pl.ds(half * D_PE + k * LANE, LANE)],
                        pe_bufs[s][2 * r + half, pl.ds(k * LANE, LANE)])

    def issue_store(c, s):
        pltpu.async_copy(f_bufs[s], out_slice(c), ssems[s])

    def wait_store(c, s):
        pltpu.make_async_copy(f_bufs[s], out_slice(c), ssems[s]).wait()

    def process(c, p):
        wait_loads(c, p)
        do_add(p)
        issue_store(c, p)

    for t in range(NSLOT):
        issue_loads(t, t)
        if t >= LOOKAHEAD:
            process(t - LOOKAHEAD, (t - LOOKAHEAD) % NSLOT)

    @pl.loop(0, NCHUNK - NSLOT, step=NSLOT)
    def _grp(i):
        for s in range(NSLOT):
            t = i + NSLOT + s
            wait_store(t - NSLOT, s)
            issue_loads(t, s)
            process(t - LOOKAHEAD, (s - LOOKAHEAD) % NSLOT)

    for t in range(NCHUNK, NCHUNK + LOOKAHEAD):
        process(t - LOOKAHEAD, (t - LOOKAHEAD) % NSLOT)
    for s in range(NSLOT):
        wait_store(NCHUNK - NSLOT + s, (NCHUNK - NSLOT + s) % NSLOT)


@jax.jit
def _pe_add(f, idx, table):
    mesh = plsc.VectorSubcoreMesh(core_axis_name="c", subcore_axis_name="s")
    return pl.kernel(
        _pe_add_kernel,
        out_type=jax.ShapeDtypeStruct((B, L, D_MODEL), jnp.float32),
        mesh=mesh,
        scratch_types=(
            [pltpu.VMEM((POS_PER_WORKER * 2,), jnp.int32)]
            + [pltpu.VMEM((CHUNK, D_MODEL), jnp.float32)] * NSLOT
            + [pltpu.VMEM((GROWS, D_PE), jnp.float32)] * NSLOT
            + [pltpu.SemaphoreType.DMA] * (3 * NSLOT)
        ),
    )(f, idx, table)


def kernel(f, rank, pe):
    idx = rank.astype(jnp.int32).reshape(B, L * 2)
    table = pe.reshape(MAX_LEN, D_PE)
    return _pe_add(f, idx, table)
